# Initial kernel scaffold; baseline (speedup 1.0000x reference)
import jax
import jax.numpy as jnp
from jax.experimental import pallas as pl


def kernel(x, weight):
    raise NotImplementedError("write your pallas kernel here")


# speedup vs baseline: 1.0798x; 1.0798x over previous
import functools

import jax
import jax.numpy as jnp
from jax import lax
from jax.experimental import pallas as pl
from jax.experimental.pallas import tpu as pltpu
from jax.experimental.pallas import tpu_sc as plsc

NUM_EMBED = 1000000
EMBED_DIM = 32
BATCH = 16384
HIST = 50
B_TOTAL = BATCH * HIST

NC = 2
NS = 16
NW = NC * NS
B_PER_W = B_TOTAL // NW
CHUNK = 128
N_CHUNK = B_PER_W // CHUNK
N_PAIR = N_CHUNK // 2

_mesh = plsc.VectorSubcoreMesh(core_axis_name="c", subcore_axis_name="s")


@functools.partial(
    pl.kernel,
    mesh=_mesh,
    out_type=jax.ShapeDtypeStruct((B_TOTAL, EMBED_DIM), jnp.float32),
    scratch_types=[
        pltpu.VMEM((N_CHUNK, CHUNK), jnp.int32),
        pltpu.VMEM((CHUNK, EMBED_DIM), jnp.float32),
        pltpu.VMEM((CHUNK, EMBED_DIM), jnp.float32),
        pltpu.SemaphoreType.DMA,
        pltpu.SemaphoreType.DMA,
    ],
    compiler_params=pltpu.CompilerParams(use_tc_tiling_on_sc=False),
)
def _gather_kernel(table_hbm, idx_hbm, out_hbm, idx_v, buf0, buf1, sem0, sem1):
    wid = lax.axis_index("s") * NC + lax.axis_index("c")
    base = wid * B_PER_W

    pltpu.sync_copy(idx_hbm.at[wid], idx_v)

    def start(j, buf, sem):
        pltpu.async_copy(table_hbm.at[idx_v.at[j]], buf, sem)

    def wait(j, buf, sem):
        pltpu.make_async_copy(table_hbm.at[idx_v.at[j]], buf, sem).wait()

    def write(j, buf):
        pltpu.sync_copy(buf, out_hbm.at[pl.ds(base + j * CHUNK, CHUNK)])

    start(0, buf0, sem0)
    start(1, buf1, sem1)

    def body(p, carry):
        j0 = 2 * p
        wait(j0, buf0, sem0)
        write(j0, buf0)
        start(j0 + 2, buf0, sem0)
        wait(j0 + 1, buf1, sem1)
        write(j0 + 1, buf1)
        start(j0 + 3, buf1, sem1)
        return carry

    lax.fori_loop(0, N_PAIR - 1, body, 0)

    j0 = N_CHUNK - 2
    wait(j0, buf0, sem0)
    write(j0, buf0)
    wait(j0 + 1, buf1, sem1)
    write(j0 + 1, buf1)


def kernel(x, weight):
    idx = x.astype(jnp.int32).reshape(NW, N_CHUNK, CHUNK)
    out = _gather_kernel(weight, idx)
    return out.reshape(BATCH, HIST, EMBED_DIM)


# speedup vs baseline: 1.1138x; 1.0316x over previous
import functools

import jax
import jax.numpy as jnp
from jax import lax
from jax.experimental import pallas as pl
from jax.experimental.pallas import tpu as pltpu
from jax.experimental.pallas import tpu_sc as plsc

NUM_EMBED = 1000000
EMBED_DIM = 32
BATCH = 16384
HIST = 50
B_TOTAL = BATCH * HIST

NC = 2
NS = 16
NW = NC * NS
B_PER_W = B_TOTAL // NW
CHUNK = 512
N_CHUNK = B_PER_W // CHUNK
NBUF = 5
N_ROUND = N_CHUNK // NBUF

_mesh = plsc.VectorSubcoreMesh(core_axis_name="c", subcore_axis_name="s")


@functools.partial(
    pl.kernel,
    mesh=_mesh,
    out_type=jax.ShapeDtypeStruct((B_TOTAL, EMBED_DIM), jnp.float32),
    scratch_types=[
        pltpu.VMEM((B_PER_W,), jnp.int32),
        [pltpu.VMEM((CHUNK, EMBED_DIM), jnp.float32) for _ in range(NBUF)],
        [pltpu.SemaphoreType.DMA for _ in range(NBUF)],
    ],
    compiler_params=pltpu.CompilerParams(use_tc_tiling_on_sc=False),
)
def _gather_kernel(table_hbm, idx_hbm, out_hbm, idx_v, bufs, sems):
    wid = lax.axis_index("s") * NC + lax.axis_index("c")
    base = wid * B_PER_W

    pltpu.sync_copy(idx_hbm.at[wid], idx_v)

    def start(j, b):
        pltpu.async_copy(
            table_hbm.at[idx_v.at[pl.ds(j * CHUNK, CHUNK)]], bufs[b], sems[b]
        )

    def wait(j, b):
        pltpu.make_async_copy(
            table_hbm.at[idx_v.at[pl.ds(j * CHUNK, CHUNK)]], bufs[b], sems[b]
        ).wait()

    def write(j, b):
        pltpu.sync_copy(bufs[b], out_hbm.at[pl.ds(base + j * CHUNK, CHUNK)])

    for b in range(NBUF):
        start(b, b)

    def round_body(p, carry):
        j0 = p * NBUF
        for b in range(NBUF):
            wait(j0 + b, b)
            write(j0 + b, b)
            start(j0 + b + NBUF, b)
        return carry

    lax.fori_loop(0, N_ROUND - 1, round_body, 0)

    j0 = (N_ROUND - 1) * NBUF
    for b in range(NBUF):
        wait(j0 + b, b)
        write(j0 + b, b)


def kernel(x, weight):
    idx = x.astype(jnp.int32).reshape(NW, B_PER_W)
    out = _gather_kernel(weight, idx)
    return out.reshape(BATCH, HIST, EMBED_DIM)


# speedup vs baseline: 1.4894x; 1.3371x over previous
import functools

import jax
import jax.numpy as jnp
from jax import lax
from jax.experimental import pallas as pl
from jax.experimental.pallas import tpu as pltpu
from jax.experimental.pallas import tpu_sc as plsc

NUM_EMBED = 1000000
EMBED_DIM = 32
BATCH = 16384
HIST = 50

NC = 2
NS = 16
NW = NC * NS
BBLK = BATCH // NW
NBUF = 2

_mesh = plsc.VectorSubcoreMesh(core_axis_name="c", subcore_axis_name="s")


@functools.partial(
    pl.kernel,
    mesh=_mesh,
    out_type=jax.ShapeDtypeStruct((HIST, EMBED_DIM, BATCH), jnp.float32),
    scratch_types=[
        pltpu.VMEM((HIST, BBLK), jnp.int32),
        [pltpu.VMEM((BBLK, EMBED_DIM), jnp.float32) for _ in range(NBUF)],
        [pltpu.VMEM((EMBED_DIM, BBLK), jnp.float32) for _ in range(NBUF)],
        [pltpu.SemaphoreType.DMA for _ in range(NBUF)],
        [pltpu.SemaphoreType.DMA for _ in range(NBUF)],
    ],
    compiler_params=pltpu.CompilerParams(
        use_tc_tiling_on_sc=False, needs_layout_passes=False
    ),
)
def _gather_kernel(
    table_hbm, xt_hbm, out_hbm, idx_v, gbufs, tbufs, g_sems, w_sems
):
    wid = lax.axis_index("s") * NC + lax.axis_index("c")
    b0 = wid * BBLK

    pltpu.sync_copy(xt_hbm.at[:, pl.ds(b0, BBLK)], idx_v)

    iota16 = lax.iota(jnp.int32, 16)

    def start_g(h, b):
        pltpu.async_copy(table_hbm.at[idx_v.at[h]], gbufs[b], g_sems[b])

    def wait_g(h, b):
        pltpu.make_async_copy(
            table_hbm.at[idx_v.at[h]], gbufs[b], g_sems[b]
        ).wait()

    def transpose(b):
        def f_body(f, carry):
            colf = jnp.full((16,), f, jnp.int32)
            for g in range(BBLK // 16):
                v = plsc.load_gather(gbufs[b], [iota16 + g * 16, colf])
                plsc.store_scatter(tbufs[b], [colf, iota16 + g * 16], v)
            return carry

        lax.fori_loop(0, EMBED_DIM, f_body, 0)

    def start_w(h, b):
        pltpu.async_copy(
            tbufs[b], out_hbm.at[h, :, pl.ds(b0, BBLK)], w_sems[b]
        )

    def wait_w(h, b):
        pltpu.make_async_copy(
            tbufs[b], out_hbm.at[h, :, pl.ds(b0, BBLK)], w_sems[b]
        ).wait()

    for b in range(NBUF):
        start_g(b, b)

    for b in range(NBUF):
        wait_g(b, b)
        transpose(b)
        start_w(b, b)
        start_g(b + NBUF, b)

    def round_body(p, carry):
        for b in range(NBUF):
            h = p * NBUF + b
            wait_g(h, b)
            wait_w(h - NBUF, b)
            transpose(b)
            start_w(h, b)
            start_g(h + NBUF, b)
        return carry

    lax.fori_loop(1, HIST // NBUF - 1, round_body, 0)

    for b in range(NBUF):
        h = HIST - NBUF + b
        wait_g(h, b)
        wait_w(h - NBUF, b)
        transpose(b)
        start_w(h, b)
    for b in range(NBUF):
        wait_w(HIST - NBUF + b, b)


def kernel(x, weight):
    xt = x.astype(jnp.int32).T
    out_t = _gather_kernel(weight, xt)
    return jnp.transpose(out_t, (2, 0, 1))


# speedup vs baseline: 2.2811x; 1.5316x over previous
import functools

import jax
import jax.numpy as jnp
from jax import lax
from jax.experimental import pallas as pl
from jax.experimental.pallas import tpu as pltpu
from jax.experimental.pallas import tpu_sc as plsc

NUM_EMBED = 1000000
EMBED_DIM = 32
BATCH = 16384
HIST = 50

NC = 2
NS = 16
NW = NC * NS
BBLK = BATCH // NW
NBUF = 2

_mesh = plsc.VectorSubcoreMesh(core_axis_name="c", subcore_axis_name="s")


@functools.partial(
    pl.kernel,
    mesh=_mesh,
    out_type=jax.ShapeDtypeStruct((HIST, EMBED_DIM, BATCH), jnp.float32),
    scratch_types=[
        pltpu.VMEM((HIST, BBLK), jnp.int32),
        [pltpu.VMEM((BBLK, EMBED_DIM), jnp.float32) for _ in range(NBUF)],
        [pltpu.VMEM((EMBED_DIM, BBLK), jnp.float32) for _ in range(NBUF)],
        [pltpu.SemaphoreType.DMA for _ in range(NBUF)],
        [pltpu.SemaphoreType.DMA for _ in range(NBUF)],
    ],
    compiler_params=pltpu.CompilerParams(
        use_tc_tiling_on_sc=False, needs_layout_passes=False
    ),
)
def _gather_kernel(
    table_hbm, xt_hbm, out_hbm, idx_v, gbufs, tbufs, g_sems, w_sems
):
    wid = lax.axis_index("s") * NC + lax.axis_index("c")
    b0 = wid * BBLK

    pltpu.sync_copy(xt_hbm.at[:, pl.ds(b0, BBLK)], idx_v)

    iota16 = lax.iota(jnp.int32, 16)

    def start_g(h, b):
        pltpu.async_copy(table_hbm.at[idx_v.at[h]], gbufs[b], g_sems[b])

    def wait_g(h, b):
        pltpu.make_async_copy(
            table_hbm.at[idx_v.at[h]], gbufs[b], g_sems[b]
        ).wait()

    def transpose(b):
        def g_body(g, carry):
            rows = iota16 + g * 16
            for f0 in range(EMBED_DIM // 16):
                for d in range(16):
                    cols = f0 * 16 + lax.bitwise_and(iota16 + d, 15)
                    v = plsc.load_gather(gbufs[b], [rows, cols])
                    plsc.store_scatter(tbufs[b], [cols, rows], v)
            return carry

        lax.fori_loop(0, BBLK // 16, g_body, 0)

    def start_w(h, b):
        pltpu.async_copy(
            tbufs[b], out_hbm.at[h, :, pl.ds(b0, BBLK)], w_sems[b]
        )

    def wait_w(h, b):
        pltpu.make_async_copy(
            tbufs[b], out_hbm.at[h, :, pl.ds(b0, BBLK)], w_sems[b]
        ).wait()

    for b in range(NBUF):
        start_g(b, b)

    for b in range(NBUF):
        wait_g(b, b)
        transpose(b)
        start_w(b, b)
        start_g(b + NBUF, b)

    def round_body(p, carry):
        for b in range(NBUF):
            h = p * NBUF + b
            wait_g(h, b)
            wait_w(h - NBUF, b)
            transpose(b)
            start_w(h, b)
            start_g(h + NBUF, b)
        return carry

    lax.fori_loop(1, HIST // NBUF - 1, round_body, 0)

    for b in range(NBUF):
        h = HIST - NBUF + b
        wait_g(h, b)
        wait_w(h - NBUF, b)
        transpose(b)
        start_w(h, b)
    for b in range(NBUF):
        wait_w(HIST - NBUF + b, b)


def kernel(x, weight):
    xt = x.astype(jnp.int32).T
    out_t = _gather_kernel(weight, xt)
    return jnp.transpose(out_t, (2, 0, 1))


# speedup vs baseline: 2.4702x; 1.0829x over previous
import functools

import jax
import jax.numpy as jnp
from jax import lax
from jax.experimental import pallas as pl
from jax.experimental.pallas import tpu as pltpu
from jax.experimental.pallas import tpu_sc as plsc

NUM_EMBED = 1000000
EMBED_DIM = 32
BATCH = 16384
HIST = 50

NC = 2
NS = 16
NW = NC * NS
BBLK = BATCH // NW
CHUNK = 256
N_CHUNK = HIST * (BBLK // CHUNK)
NBUF = 2

SUP_ROWS = NUM_EMBED // 4

_mesh = plsc.VectorSubcoreMesh(core_axis_name="c", subcore_axis_name="s")


@functools.partial(
    pl.kernel,
    mesh=_mesh,
    out_type=jax.ShapeDtypeStruct((HIST, EMBED_DIM, BATCH), jnp.float32),
    scratch_types=[
        pltpu.VMEM((HIST, BBLK), jnp.int32),
        [pltpu.VMEM((CHUNK, 128), jnp.float32) for _ in range(NBUF)],
        [pltpu.VMEM((EMBED_DIM, CHUNK), jnp.float32) for _ in range(NBUF)],
        [pltpu.VMEM((2, 128), jnp.int32) for _ in range(NBUF)],
        [pltpu.VMEM((CHUNK,), jnp.int32) for _ in range(NBUF)],
        [pltpu.SemaphoreType.DMA for _ in range(NBUF)],
        [pltpu.SemaphoreType.DMA for _ in range(NBUF)],
    ],
    compiler_params=pltpu.CompilerParams(needs_layout_passes=False),
)
def _gather_kernel(
    table_hbm, xt_hbm, out_hbm, idx_v, gbufs, tbufs, sup_idx, qoffs,
    g_sems, w_sems,
):
    wid = lax.axis_index("s") * NC + lax.axis_index("c")
    b0 = wid * BBLK

    pltpu.sync_copy(xt_hbm.at[:, pl.ds(b0, BBLK)], idx_v)

    iota16 = lax.iota(jnp.int32, 16)

    def compute(j, b):
        h = lax.div(j, 2)
        off = lax.rem(j, 2) * CHUNK
        for t in range(CHUNK // 16):
            v = idx_v[h, pl.ds(off + t * 16, 16)]
            sup_idx[b][t // 8, pl.ds((t % 8) * 16, 16)] = (
                lax.shift_right_logical(v, 2)
            )
            qoffs[b][pl.ds(t * 16, 16)] = lax.bitwise_and(v, 3) * 32

    def start_g(b):
        for k in range(2):
            pltpu.async_copy(
                table_hbm.at[sup_idx[b].at[k]],
                gbufs[b].at[pl.ds(k * 128, 128)],
                g_sems[b],
            )

    def wait_g(b):
        for k in range(2):
            pltpu.make_async_copy(
                table_hbm.at[sup_idx[b].at[k]],
                gbufs[b].at[pl.ds(k * 128, 128)],
                g_sems[b],
            ).wait()

    def transpose(b):
        def c_body(c, carry):
            rows = iota16 + c * 16
            qv = qoffs[b][pl.ds(c * 16, 16)]
            for f0 in range(EMBED_DIM // 16):
                for d in range(16):
                    fvec = f0 * 16 + lax.bitwise_and(iota16 + d, 15)
                    v = plsc.load_gather(gbufs[b], [rows, qv + fvec])
                    plsc.store_scatter(tbufs[b], [fvec, rows], v)
            return carry

        lax.fori_loop(0, CHUNK // 16, c_body, 0)

    def start_w(j, b):
        h = lax.div(j, 2)
        off = lax.rem(j, 2) * CHUNK
        pltpu.async_copy(
            tbufs[b], out_hbm.at[h, :, pl.ds(b0 + off, CHUNK)], w_sems[b]
        )

    def wait_w(j, b):
        h = lax.div(j, 2)
        off = lax.rem(j, 2) * CHUNK
        pltpu.make_async_copy(
            tbufs[b], out_hbm.at[h, :, pl.ds(b0 + off, CHUNK)], w_sems[b]
        ).wait()

    for b in range(NBUF):
        compute(b, b)
        start_g(b)

    for b in range(NBUF):
        wait_g(b)
        transpose(b)
        start_w(b, b)
        compute(b + NBUF, b)
        start_g(b)

    def round_body(p, carry):
        for b in range(NBUF):
            j = p * NBUF + b
            wait_g(b)
            wait_w(j - NBUF, b)
            transpose(b)
            start_w(j, b)
            compute(j + NBUF, b)
            start_g(b)
        return carry

    lax.fori_loop(1, N_CHUNK // NBUF - 1, round_body, 0)

    for b in range(NBUF):
        j = N_CHUNK - NBUF + b
        wait_g(b)
        wait_w(j - NBUF, b)
        transpose(b)
        start_w(j, b)
    for b in range(NBUF):
        wait_w(N_CHUNK - NBUF + b, b)


def kernel(x, weight):
    xt = x.astype(jnp.int32).T
    table = weight.reshape(SUP_ROWS, 128)
    out_t = _gather_kernel(table, xt)
    return jnp.transpose(out_t, (2, 0, 1))
